# baseline (speedup 1.0000x reference)
import jax
import jax.numpy as jnp
from jax import lax
from jax.experimental import pallas as pl
from jax.experimental.pallas import tpu as pltpu
from jax.experimental.pallas import tpu_sc as plsc

NUM_ROWS = 1000000
BATCH = 16384
BAG_LEN = 20
DIM = 32
BASE_DIM = 128

NC = 2
NS = 16
NW = NC * NS
BAGS_PW = BATCH // NW
ROWS_PW = BAGS_PW * BAG_LEN
IDX_MINOR = 128
IDX_MAJOR = ROWS_PW // IDX_MINOR

CHUNK_BAGS = 32
CHUNK_ROWS = CHUNK_BAGS * BAG_LEN
GATHERS_PER_CHUNK = CHUNK_ROWS // IDX_MINOR
N_CHUNKS = BAGS_PW // CHUNK_BAGS

PACK = 4
PROWS_PW = BAGS_PW // PACK

TSP_S = 4096
TSP_SHIFT = TSP_S.bit_length() - 1
TSP_BLK = PACK * TSP_S
NUM_ROWS_PAD = 1048576


def _sc_pool_kernel(inp_hbm, table_hbm, pooled_hbm, idx_v, rows_a, rows_b,
                    pooled_v, sem_a, sem_b):
    wid = lax.axis_index("s") * NC + lax.axis_index("c")
    pltpu.sync_copy(inp_hbm.at[pl.ds(wid * ROWS_PW, ROWS_PW)], idx_v)

    def sigma_body(j, _):
        t = idx_v[pl.ds(j * 16, 16)]
        idx_v[pl.ds(j * 16, 16)] = (
            (t & jnp.int32(-TSP_BLK))
            | ((t & jnp.int32(TSP_S - 1)) << 2)
            | ((t >> TSP_SHIFT) & jnp.int32(3))
        )
        return 0

    lax.fori_loop(0, ROWS_PW // 16, sigma_body, 0)

    bufs = (rows_a, rows_b)
    sems = (sem_a, sem_b)
    descs = [None] * N_CHUNKS

    def start(g):
        buf = bufs[g % 2]
        sem = sems[g % 2]
        descs[g] = [
            pltpu.async_copy(
                table_hbm.at[idx_v.at[pl.ds(g * CHUNK_ROWS, CHUNK_ROWS)]],
                buf,
                sem,
            )
        ]

    def pool(g):
        buf = bufs[g % 2]

        def bag_body(b, _):
            r0 = b * BAG_LEN
            acc0 = buf[r0, pl.ds(0, 16)]
            acc1 = buf[r0, pl.ds(16, 16)]
            for l in range(1, BAG_LEN):
                acc0 = acc0 + buf[r0 + l, pl.ds(0, 16)]
                acc1 = acc1 + buf[r0 + l, pl.ds(16, 16)]
            ob = g * CHUNK_BAGS + b
            pooled_v[ob, pl.ds(0, 16)] = acc0
            pooled_v[ob, pl.ds(16, 16)] = acc1
            return 0

        lax.fori_loop(0, CHUNK_BAGS, bag_body, 0)

    start(0)
    for g in range(N_CHUNKS):
        if g + 1 < N_CHUNKS:
            start(g + 1)
        for d in descs[g]:
            d.wait()
        pool(g)

    pltpu.sync_copy(
        pooled_v,
        pooled_hbm.at[pl.ds((wid // PACK) * BAGS_PW, BAGS_PW),
                      pl.ds((wid % PACK) * DIM, DIM)],
    )


def _sc_pool(inp3, table):
    mesh = plsc.VectorSubcoreMesh(
        core_axis_name="c", subcore_axis_name="s", num_cores=NC, num_subcores=NS
    )
    return pl.kernel(
        _sc_pool_kernel,
        out_type=jax.ShapeDtypeStruct((BATCH // PACK, PACK * DIM), jnp.float32),
        mesh=mesh,
        scratch_types=[
            pltpu.VMEM((ROWS_PW,), jnp.int32),
            pltpu.VMEM((CHUNK_ROWS, DIM), jnp.float32),
            pltpu.VMEM((CHUNK_ROWS, DIM), jnp.float32),
            pltpu.VMEM((BAGS_PW, DIM), jnp.float32),
            pltpu.SemaphoreType.DMA,
            pltpu.SemaphoreType.DMA,
        ],
        compiler_params=pltpu.CompilerParams(use_tc_tiling_on_sc=False),
    )(inp3, table)


def _transpose_body(x_ref, eye_ref, o_ref):
    acc = None
    for c in range(PACK):
        xc = x_ref[:, pl.ds(c * TSP_S, TSP_S)]
        xh = xc.astype(jnp.bfloat16)
        xl = (xc - xh.astype(jnp.float32)).astype(jnp.bfloat16)
        ec = eye_ref[pl.ds(c * DIM, DIM), :]
        for part in (xh, xl):
            yc = lax.dot_general(
                part, ec, (((0,), (0,)), ((), ())),
                preferred_element_type=jnp.float32,
            )
            acc = yc if acc is None else acc + yc
    o_ref[...] = acc


def _tc_transpose(table_t):
    grid = (NUM_ROWS_PAD // TSP_BLK,)
    eye = jnp.eye(PACK * DIM, dtype=jnp.bfloat16)
    return pl.pallas_call(
        _transpose_body,
        grid=grid,
        in_specs=[
            pl.BlockSpec((DIM, TSP_BLK),
                         lambda i: (0, jnp.minimum(i, (NUM_ROWS - 1) // TSP_BLK))),
            pl.BlockSpec((PACK * DIM, PACK * DIM), lambda i: (0, 0)),
        ],
        out_specs=pl.BlockSpec((TSP_S, PACK * DIM), lambda i: (i, 0)),
        out_shape=jax.ShapeDtypeStruct((NUM_ROWS_PAD // PACK, PACK * DIM), jnp.float32),
    )(table_t, eye)


def _proj_body(x_ref, w_ref, o_ref):
    blk = x_ref.shape[0]
    for i in range(PACK):
        xi = x_ref[:, pl.ds(i * DIM, DIM)]
        yi = lax.dot_general(
            xi, w_ref[...], (((1,), (1,)), ((), ())),
            preferred_element_type=jnp.float32,
        )
        o_ref[pl.ds(i * blk, blk), :] = yi


def _tc_proj(pooled_packed, w_proj):
    blk = 512
    grid = (BATCH // (blk * PACK),)
    return pl.pallas_call(
        _proj_body,
        grid=grid,
        in_specs=[
            pl.BlockSpec((blk, PACK * DIM), lambda i: (i, 0)),
            pl.BlockSpec((BASE_DIM, DIM), lambda i: (0, 0)),
        ],
        out_specs=pl.BlockSpec((blk * PACK, BASE_DIM), lambda i: (i, 0)),
        out_shape=jax.ShapeDtypeStruct((BATCH, BASE_DIM), jnp.float32),
    )(pooled_packed, w_proj)


def kernel(input, table, W_proj):
    inp3 = input.reshape(BATCH * BAG_LEN)
    table_rm = _tc_transpose(table.T).reshape(NUM_ROWS_PAD, DIM)
    pooled = _sc_pool(inp3, table_rm)
    return _tc_proj(pooled, W_proj)


# speedup vs baseline: 1.2046x; 1.2046x over previous
import jax
import jax.numpy as jnp
from jax import lax
from jax.experimental import pallas as pl
from jax.experimental.pallas import tpu as pltpu
from jax.experimental.pallas import tpu_sc as plsc

NUM_ROWS = 1000000
BATCH = 16384
BAG_LEN = 20
DIM = 32
BASE_DIM = 128

NC = 2
NS = 16
NW = NC * NS
BAGS_PW = BATCH // NW
ROWS_PW = BAGS_PW * BAG_LEN
IDX_MINOR = 128
IDX_MAJOR = ROWS_PW // IDX_MINOR

CHUNK_BAGS = 32
CHUNK_ROWS = CHUNK_BAGS * BAG_LEN
GATHERS_PER_CHUNK = CHUNK_ROWS // IDX_MINOR
N_CHUNKS = BAGS_PW // CHUNK_BAGS

PACK = 4
PROWS_PW = BAGS_PW // PACK

TSP_S = 4096
TSP_SHIFT = TSP_S.bit_length() - 1
TSP_BLK = PACK * TSP_S
NUM_ROWS_PAD = 1048576


def _sc_pool_kernel(inp_hbm, table_hbm, pooled_hbm, idx_v, rows_a, rows_b,
                    pooled_v, sem_a, sem_b):
    wid = lax.axis_index("s") * NC + lax.axis_index("c")
    pltpu.sync_copy(inp_hbm.at[pl.ds(wid * ROWS_PW, ROWS_PW)], idx_v)

    def sigma_body(j, _):
        t = idx_v[pl.ds(j * 16, 16)]
        idx_v[pl.ds(j * 16, 16)] = (
            (t & jnp.int32(-TSP_BLK))
            | ((t & jnp.int32(TSP_S - 1)) << 2)
            | ((t >> TSP_SHIFT) & jnp.int32(3))
        )
        return 0

    lax.fori_loop(0, ROWS_PW // 16, sigma_body, 0)

    bufs = (rows_a, rows_b)
    sems = (sem_a, sem_b)
    descs = [None] * N_CHUNKS

    def start(g):
        buf = bufs[g % 2]
        sem = sems[g % 2]
        descs[g] = [
            pltpu.async_copy(
                table_hbm.at[idx_v.at[pl.ds(g * CHUNK_ROWS, CHUNK_ROWS)]],
                buf,
                sem,
            )
        ]

    def pool(g):
        buf = bufs[g % 2]

        def bag_body(b, _):
            r0 = b * BAG_LEN
            acc0 = buf[r0, pl.ds(0, 16)]
            acc1 = buf[r0, pl.ds(16, 16)]
            for l in range(1, BAG_LEN):
                acc0 = acc0 + buf[r0 + l, pl.ds(0, 16)]
                acc1 = acc1 + buf[r0 + l, pl.ds(16, 16)]
            ob = g * CHUNK_BAGS + b
            pooled_v[ob, pl.ds(0, 16)] = acc0
            pooled_v[ob, pl.ds(16, 16)] = acc1
            return 0

        lax.fori_loop(0, CHUNK_BAGS, bag_body, 0)

    start(0)
    for g in range(N_CHUNKS):
        if g + 1 < N_CHUNKS:
            start(g + 1)
        for d in descs[g]:
            d.wait()
        pool(g)

    pltpu.sync_copy(
        pooled_v,
        pooled_hbm.at[pl.ds((wid // PACK) * BAGS_PW, BAGS_PW),
                      pl.ds((wid % PACK) * DIM, DIM)],
    )


def _sc_pool(inp3, table):
    mesh = plsc.VectorSubcoreMesh(
        core_axis_name="c", subcore_axis_name="s", num_cores=NC, num_subcores=NS
    )
    return pl.kernel(
        _sc_pool_kernel,
        out_type=jax.ShapeDtypeStruct((BATCH // PACK, PACK * DIM), jnp.float32),
        mesh=mesh,
        scratch_types=[
            pltpu.VMEM((ROWS_PW,), jnp.int32),
            pltpu.VMEM((CHUNK_ROWS, DIM), jnp.float32),
            pltpu.VMEM((CHUNK_ROWS, DIM), jnp.float32),
            pltpu.VMEM((BAGS_PW, DIM), jnp.float32),
            pltpu.SemaphoreType.DMA,
            pltpu.SemaphoreType.DMA,
        ],
        compiler_params=pltpu.CompilerParams(use_tc_tiling_on_sc=False),
    )(inp3, table)


def _transpose_body(x_ref, eye_ref, o_ref):
    acc = None
    for c in range(PACK):
        xc = x_ref[:, pl.ds(c * TSP_S, TSP_S)]
        xh = xc.astype(jnp.bfloat16)
        xl = (xc - xh.astype(jnp.float32)).astype(jnp.bfloat16)
        ec = eye_ref[pl.ds(c * DIM, DIM), :]
        for part in (xh, xl):
            yc = lax.dot_general(
                part, ec, (((0,), (0,)), ((), ())),
                preferred_element_type=jnp.float32,
            )
            acc = yc if acc is None else acc + yc
    o_ref[...] = acc


def _transpose_body_bf16(x_ref, eye_ref, o_ref):
    acc = None
    for c in range(PACK):
        xc = x_ref[:, pl.ds(c * TSP_S, TSP_S)].astype(jnp.bfloat16)
        yc = lax.dot_general(
            xc, eye_ref[pl.ds(c * DIM, DIM), :], (((0,), (0,)), ((), ())),
            preferred_element_type=jnp.float32,
        )
        acc = yc if acc is None else acc + yc
    o_ref[...] = acc


def _tc_transpose(table_t):
    grid = (NUM_ROWS_PAD // TSP_BLK,)
    eye = jnp.eye(PACK * DIM, dtype=jnp.bfloat16)
    return pl.pallas_call(
        _transpose_body_bf16,
        grid=grid,
        in_specs=[
            pl.BlockSpec((DIM, TSP_BLK),
                         lambda i: (0, jnp.minimum(i, (NUM_ROWS - 1) // TSP_BLK))),
            pl.BlockSpec((PACK * DIM, PACK * DIM), lambda i: (0, 0)),
        ],
        out_specs=pl.BlockSpec((TSP_S, PACK * DIM), lambda i: (i, 0)),
        out_shape=jax.ShapeDtypeStruct((NUM_ROWS_PAD // PACK, PACK * DIM), jnp.float32),
    )(table_t, eye)


def _proj_body(x_ref, w_ref, o_ref):
    blk = x_ref.shape[0]
    for i in range(PACK):
        xi = x_ref[:, pl.ds(i * DIM, DIM)]
        yi = lax.dot_general(
            xi, w_ref[...], (((1,), (1,)), ((), ())),
            preferred_element_type=jnp.float32,
        )
        o_ref[pl.ds(i * blk, blk), :] = yi


def _tc_proj(pooled_packed, w_proj):
    blk = 512
    grid = (BATCH // (blk * PACK),)
    return pl.pallas_call(
        _proj_body,
        grid=grid,
        in_specs=[
            pl.BlockSpec((blk, PACK * DIM), lambda i: (i, 0)),
            pl.BlockSpec((BASE_DIM, DIM), lambda i: (0, 0)),
        ],
        out_specs=pl.BlockSpec((blk * PACK, BASE_DIM), lambda i: (i, 0)),
        out_shape=jax.ShapeDtypeStruct((BATCH, BASE_DIM), jnp.float32),
    )(pooled_packed, w_proj)


def kernel(input, table, W_proj):
    inp3 = input.reshape(BATCH * BAG_LEN)
    table_rm = _tc_transpose(table.T).reshape(NUM_ROWS_PAD, DIM)
    pooled = _sc_pool(inp3, table_rm)
    return _tc_proj(pooled, W_proj)


# speedup vs baseline: 1.3407x; 1.1130x over previous
import jax
import jax.numpy as jnp
from jax import lax
from jax.experimental import pallas as pl
from jax.experimental.pallas import tpu as pltpu
from jax.experimental.pallas import tpu_sc as plsc

NUM_ROWS = 1000000
BATCH = 16384
BAG_LEN = 20
DIM = 32
BASE_DIM = 128

NC = 2
NS = 16
NW = NC * NS
BAGS_PW = BATCH // NW
ROWS_PW = BAGS_PW * BAG_LEN
IDX_MINOR = 128
IDX_MAJOR = ROWS_PW // IDX_MINOR

CHUNK_BAGS = 32
CHUNK_ROWS = CHUNK_BAGS * BAG_LEN
GATHERS_PER_CHUNK = CHUNK_ROWS // IDX_MINOR
N_CHUNKS = BAGS_PW // CHUNK_BAGS

PACK = 4
PROWS_PW = BAGS_PW // PACK

TSP_S = 8192
TSP_SHIFT = TSP_S.bit_length() - 1
TSP_BLK = PACK * TSP_S
NUM_ROWS_PAD = 1048576


def _sc_pool_kernel(inp_hbm, table_hbm, pooled_hbm, idx_v, rows_a, rows_b,
                    pooled_v, sem_a, sem_b):
    wid = lax.axis_index("s") * NC + lax.axis_index("c")
    pltpu.sync_copy(inp_hbm.at[pl.ds(wid * ROWS_PW, ROWS_PW)], idx_v)

    def sigma_body(j, _):
        t = idx_v[pl.ds(j * 16, 16)]
        idx_v[pl.ds(j * 16, 16)] = (
            (t & jnp.int32(-TSP_BLK))
            | ((t & jnp.int32(TSP_S - 1)) << 2)
            | ((t >> TSP_SHIFT) & jnp.int32(3))
        )
        return 0

    lax.fori_loop(0, ROWS_PW // 16, sigma_body, 0)

    bufs = (rows_a, rows_b)
    sems = (sem_a, sem_b)
    descs = [None] * N_CHUNKS

    def start(g):
        buf = bufs[g % 2]
        sem = sems[g % 2]
        descs[g] = [
            pltpu.async_copy(
                table_hbm.at[idx_v.at[pl.ds(g * CHUNK_ROWS, CHUNK_ROWS)]],
                buf,
                sem,
            )
        ]

    def pool(g):
        buf = bufs[g % 2]

        def bag_body(b, _):
            r0 = b * BAG_LEN
            acc0 = buf[r0, pl.ds(0, 16)]
            acc1 = buf[r0, pl.ds(16, 16)]
            for l in range(1, BAG_LEN):
                acc0 = acc0 + buf[r0 + l, pl.ds(0, 16)]
                acc1 = acc1 + buf[r0 + l, pl.ds(16, 16)]
            ob = g * CHUNK_BAGS + b
            pooled_v[ob, pl.ds(0, 16)] = acc0
            pooled_v[ob, pl.ds(16, 16)] = acc1
            return 0

        lax.fori_loop(0, CHUNK_BAGS, bag_body, 0)

    start(0)
    for g in range(N_CHUNKS):
        if g + 1 < N_CHUNKS:
            start(g + 1)
        for d in descs[g]:
            d.wait()
        pool(g)

    pltpu.sync_copy(
        pooled_v,
        pooled_hbm.at[pl.ds((wid // PACK) * BAGS_PW, BAGS_PW),
                      pl.ds((wid % PACK) * DIM, DIM)],
    )


def _sc_pool(inp3, table):
    mesh = plsc.VectorSubcoreMesh(
        core_axis_name="c", subcore_axis_name="s", num_cores=NC, num_subcores=NS
    )
    return pl.kernel(
        _sc_pool_kernel,
        out_type=jax.ShapeDtypeStruct((BATCH // PACK, PACK * DIM), jnp.float32),
        mesh=mesh,
        scratch_types=[
            pltpu.VMEM((ROWS_PW,), jnp.int32),
            pltpu.VMEM((CHUNK_ROWS, DIM), jnp.float32),
            pltpu.VMEM((CHUNK_ROWS, DIM), jnp.float32),
            pltpu.VMEM((BAGS_PW, DIM), jnp.float32),
            pltpu.SemaphoreType.DMA,
            pltpu.SemaphoreType.DMA,
        ],
        compiler_params=pltpu.CompilerParams(use_tc_tiling_on_sc=False),
    )(inp3, table)


def _transpose_body(x_ref, eye_ref, o_ref):
    acc = None
    for c in range(PACK):
        xc = x_ref[:, pl.ds(c * TSP_S, TSP_S)]
        xh = xc.astype(jnp.bfloat16)
        xl = (xc - xh.astype(jnp.float32)).astype(jnp.bfloat16)
        ec = eye_ref[pl.ds(c * DIM, DIM), :]
        for part in (xh, xl):
            yc = lax.dot_general(
                part, ec, (((0,), (0,)), ((), ())),
                preferred_element_type=jnp.float32,
            )
            acc = yc if acc is None else acc + yc
    o_ref[...] = acc


def _transpose_body_bf16(x_ref, eye_ref, o_ref):
    acc = None
    for c in range(PACK):
        xc = x_ref[:, pl.ds(c * TSP_S, TSP_S)].astype(jnp.bfloat16)
        yc = lax.dot_general(
            xc, eye_ref[pl.ds(c * DIM, DIM), :], (((0,), (0,)), ((), ())),
            preferred_element_type=jnp.float32,
        )
        acc = yc if acc is None else acc + yc
    o_ref[...] = acc


def _tc_transpose(table_t):
    grid = (NUM_ROWS_PAD // TSP_BLK,)
    eye = jnp.eye(PACK * DIM, dtype=jnp.bfloat16)
    return pl.pallas_call(
        _transpose_body_bf16,
        grid=grid,
        in_specs=[
            pl.BlockSpec((DIM, TSP_BLK),
                         lambda i: (0, jnp.minimum(i, (NUM_ROWS - 1) // TSP_BLK))),
            pl.BlockSpec((PACK * DIM, PACK * DIM), lambda i: (0, 0)),
        ],
        out_specs=pl.BlockSpec((TSP_S, PACK * DIM), lambda i: (i, 0)),
        out_shape=jax.ShapeDtypeStruct((NUM_ROWS_PAD // PACK, PACK * DIM), jnp.float32),
    )(table_t, eye)


def _proj_body(x_ref, w_ref, o_ref):
    blk = x_ref.shape[0]
    for i in range(PACK):
        xi = x_ref[:, pl.ds(i * DIM, DIM)]
        yi = lax.dot_general(
            xi, w_ref[...], (((1,), (1,)), ((), ())),
            preferred_element_type=jnp.float32,
        )
        o_ref[pl.ds(i * blk, blk), :] = yi


def _tc_proj(pooled_packed, w_proj):
    blk = 512
    grid = (BATCH // (blk * PACK),)
    return pl.pallas_call(
        _proj_body,
        grid=grid,
        in_specs=[
            pl.BlockSpec((blk, PACK * DIM), lambda i: (i, 0)),
            pl.BlockSpec((BASE_DIM, DIM), lambda i: (0, 0)),
        ],
        out_specs=pl.BlockSpec((blk * PACK, BASE_DIM), lambda i: (i, 0)),
        out_shape=jax.ShapeDtypeStruct((BATCH, BASE_DIM), jnp.float32),
    )(pooled_packed, w_proj)


def kernel(input, table, W_proj):
    inp3 = input.reshape(BATCH * BAG_LEN)
    table_rm = _tc_transpose(table.T).reshape(NUM_ROWS_PAD, DIM)
    pooled = _sc_pool(inp3, table_rm)
    return _tc_proj(pooled, W_proj)


# speedup vs baseline: 1.4082x; 1.0503x over previous
import jax
import jax.numpy as jnp
from jax import lax
from jax.experimental import pallas as pl
from jax.experimental.pallas import tpu as pltpu
from jax.experimental.pallas import tpu_sc as plsc

NUM_ROWS = 1000000
BATCH = 16384
BAG_LEN = 20
DIM = 32
BASE_DIM = 128

NC = 2
NS = 16
NW = NC * NS
BAGS_PW = BATCH // NW
ROWS_PW = BAGS_PW * BAG_LEN
IDX_MINOR = 128
IDX_MAJOR = ROWS_PW // IDX_MINOR

CHUNK_BAGS = 32
CHUNK_ROWS = CHUNK_BAGS * BAG_LEN
GATHERS_PER_CHUNK = CHUNK_ROWS // IDX_MINOR
N_CHUNKS = BAGS_PW // CHUNK_BAGS

PACK = 4
PROWS_PW = BAGS_PW // PACK

TSP_S = 16384
TSP_SHIFT = TSP_S.bit_length() - 1
TSP_BLK = PACK * TSP_S
NUM_ROWS_PAD = 1048576


def _sc_pool_kernel(inp_hbm, table_hbm, pooled_hbm, idx_v, rows_a, rows_b,
                    pooled_v, sem_a, sem_b):
    wid = lax.axis_index("s") * NC + lax.axis_index("c")
    pltpu.sync_copy(inp_hbm.at[pl.ds(wid * ROWS_PW, ROWS_PW)], idx_v)

    def sigma_body(j, _):
        t = idx_v[pl.ds(j * 16, 16)]
        idx_v[pl.ds(j * 16, 16)] = (
            (t & jnp.int32(-TSP_BLK))
            | ((t & jnp.int32(TSP_S - 1)) << 2)
            | ((t >> TSP_SHIFT) & jnp.int32(3))
        )
        return 0

    lax.fori_loop(0, ROWS_PW // 16, sigma_body, 0)

    bufs = (rows_a, rows_b)
    sems = (sem_a, sem_b)
    descs = [None] * N_CHUNKS

    def start(g):
        buf = bufs[g % 2]
        sem = sems[g % 2]
        descs[g] = [
            pltpu.async_copy(
                table_hbm.at[idx_v.at[pl.ds(g * CHUNK_ROWS, CHUNK_ROWS)]],
                buf,
                sem,
            )
        ]

    def pool(g):
        buf = bufs[g % 2]

        def bag_body(b, _):
            r0 = b * BAG_LEN
            acc0 = buf[r0, pl.ds(0, 16)]
            acc1 = buf[r0, pl.ds(16, 16)]
            for l in range(1, BAG_LEN):
                acc0 = acc0 + buf[r0 + l, pl.ds(0, 16)]
                acc1 = acc1 + buf[r0 + l, pl.ds(16, 16)]
            ob = g * CHUNK_BAGS + b
            pooled_v[ob, pl.ds(0, 16)] = acc0
            pooled_v[ob, pl.ds(16, 16)] = acc1
            return 0

        lax.fori_loop(0, CHUNK_BAGS, bag_body, 0)

    start(0)
    for g in range(N_CHUNKS):
        if g + 1 < N_CHUNKS:
            start(g + 1)
        for d in descs[g]:
            d.wait()
        pool(g)

    pltpu.sync_copy(
        pooled_v,
        pooled_hbm.at[pl.ds((wid // PACK) * BAGS_PW, BAGS_PW),
                      pl.ds((wid % PACK) * DIM, DIM)],
    )


def _sc_pool(inp3, table):
    mesh = plsc.VectorSubcoreMesh(
        core_axis_name="c", subcore_axis_name="s", num_cores=NC, num_subcores=NS
    )
    return pl.kernel(
        _sc_pool_kernel,
        out_type=jax.ShapeDtypeStruct((BATCH // PACK, PACK * DIM), jnp.float32),
        mesh=mesh,
        scratch_types=[
            pltpu.VMEM((ROWS_PW,), jnp.int32),
            pltpu.VMEM((CHUNK_ROWS, DIM), jnp.float32),
            pltpu.VMEM((CHUNK_ROWS, DIM), jnp.float32),
            pltpu.VMEM((BAGS_PW, DIM), jnp.float32),
            pltpu.SemaphoreType.DMA,
            pltpu.SemaphoreType.DMA,
        ],
        compiler_params=pltpu.CompilerParams(use_tc_tiling_on_sc=False),
    )(inp3, table)


def _transpose_body(x_ref, eye_ref, o_ref):
    acc = None
    for c in range(PACK):
        xc = x_ref[:, pl.ds(c * TSP_S, TSP_S)]
        xh = xc.astype(jnp.bfloat16)
        xl = (xc - xh.astype(jnp.float32)).astype(jnp.bfloat16)
        ec = eye_ref[pl.ds(c * DIM, DIM), :]
        for part in (xh, xl):
            yc = lax.dot_general(
                part, ec, (((0,), (0,)), ((), ())),
                preferred_element_type=jnp.float32,
            )
            acc = yc if acc is None else acc + yc
    o_ref[...] = acc


def _transpose_body_bf16(x_ref, eye_ref, o_ref):
    acc = None
    for c in range(PACK):
        xc = x_ref[:, pl.ds(c * TSP_S, TSP_S)].astype(jnp.bfloat16)
        yc = lax.dot_general(
            xc, eye_ref[pl.ds(c * DIM, DIM), :], (((0,), (0,)), ((), ())),
            preferred_element_type=jnp.float32,
        )
        acc = yc if acc is None else acc + yc
    o_ref[...] = acc


def _tc_transpose(table_t):
    grid = (NUM_ROWS_PAD // TSP_BLK,)
    eye = jnp.eye(PACK * DIM, dtype=jnp.bfloat16)
    return pl.pallas_call(
        _transpose_body_bf16,
        grid=grid,
        in_specs=[
            pl.BlockSpec((DIM, TSP_BLK),
                         lambda i: (0, jnp.minimum(i, (NUM_ROWS - 1) // TSP_BLK))),
            pl.BlockSpec((PACK * DIM, PACK * DIM), lambda i: (0, 0)),
        ],
        out_specs=pl.BlockSpec((TSP_S, PACK * DIM), lambda i: (i, 0)),
        out_shape=jax.ShapeDtypeStruct((NUM_ROWS_PAD // PACK, PACK * DIM), jnp.float32),
    )(table_t, eye)


def _proj_body(x_ref, w_ref, o_ref):
    blk = x_ref.shape[0]
    for i in range(PACK):
        xi = x_ref[:, pl.ds(i * DIM, DIM)]
        yi = lax.dot_general(
            xi, w_ref[...], (((1,), (1,)), ((), ())),
            preferred_element_type=jnp.float32,
        )
        o_ref[pl.ds(i * blk, blk), :] = yi


def _tc_proj(pooled_packed, w_proj):
    blk = 512
    grid = (BATCH // (blk * PACK),)
    return pl.pallas_call(
        _proj_body,
        grid=grid,
        in_specs=[
            pl.BlockSpec((blk, PACK * DIM), lambda i: (i, 0)),
            pl.BlockSpec((BASE_DIM, DIM), lambda i: (0, 0)),
        ],
        out_specs=pl.BlockSpec((blk * PACK, BASE_DIM), lambda i: (i, 0)),
        out_shape=jax.ShapeDtypeStruct((BATCH, BASE_DIM), jnp.float32),
    )(pooled_packed, w_proj)


def kernel(input, table, W_proj):
    inp3 = input.reshape(BATCH * BAG_LEN)
    table_rm = _tc_transpose(table.T).reshape(NUM_ROWS_PAD, DIM)
    pooled = _sc_pool(inp3, table_rm)
    return _tc_proj(pooled, W_proj)
